# baseline (speedup 1.0000x reference)
import functools

import jax
import jax.numpy as jnp
from jax import lax
from jax.experimental import pallas as pl
from jax.experimental.pallas import tpu as pltpu
from jax.experimental.pallas import tpu_sc as plsc

EMB_DIM = 16
HIST = 200
IDX_MINOR = 80
ROWS_PER_CHUNK = 8
IDXROWS_PER_CHUNK = ROWS_PER_CHUNK * HIST // IDX_MINOR
TRANS_W = 1536


def _sc_detile_table(tab_t, tail128):
    vocab = tab_t.shape[1]
    aligned_v = vocab // 128 * 128
    n_full = aligned_v // TRANS_W
    rem = aligned_v - n_full * TRANS_W
    tail_start = vocab - 128
    info = plsc.get_sparse_core_info()
    nw = info.num_cores * info.num_subcores
    mesh = plsc.VectorSubcoreMesh(core_axis_name="c", subcore_axis_name="s")

    @functools.partial(
        pl.kernel,
        out_type=jax.ShapeDtypeStruct((vocab * EMB_DIM,), jnp.float32),
        mesh=mesh,
        scratch_types=[
            pltpu.VMEM((EMB_DIM, TRANS_W), jnp.float32),
            pltpu.VMEM((EMB_DIM, TRANS_W), jnp.float32),
            pltpu.VMEM((TRANS_W * EMB_DIM,), jnp.float32),
            pltpu.VMEM((TRANS_W * EMB_DIM,), jnp.float32),
            pltpu.VMEM((128, EMB_DIM), jnp.float32),
            pltpu.SemaphoreType.DMA,
            pltpu.SemaphoreType.DMA,
            pltpu.SemaphoreType.DMA,
            pltpu.SemaphoreType.DMA,
        ],
        compiler_params=pltpu.CompilerParams(use_tc_tiling_on_sc=True,
                                             needs_layout_passes=False),
    )
    def k(tab_hbm, tail_hbm, out_hbm, in_v0, in_v1, out_v0, out_v1, tail_v,
          isem0, isem1, osem0, osem1):
        wid = lax.axis_index("s") * info.num_cores + lax.axis_index("c")
        rows_i = lax.iota(jnp.int32, 16)
        in_vs = (in_v0, in_v1)
        out_vs = (out_v0, out_v1)
        isems = (isem0, isem1)
        osems = (osem0, osem1)
        n_mine = (n_full - 1 - wid) // nw + 1

        def c0_of(t):
            return (wid + t * nw) * TRANS_W

        def fire_in(t, b):
            pltpu.async_copy(tab_hbm.at[:, pl.ds(c0_of(t), TRANS_W)],
                             in_vs[b], isems[b])

        def transpose_chunk(b, width):
            def blk(i, carry, b=b):
                c0 = i * 16
                rot = rows_i
                for _ in range(16):
                    colv = rot + c0
                    gvals = plsc.load_gather(in_vs[b], [rows_i, colv])
                    sidx = jnp.left_shift(colv, 4) + rows_i
                    plsc.store_scatter(out_vs[b], [sidx], gvals)
                    rot = jnp.bitwise_and(rot + 1, 15)
                return carry

            lax.fori_loop(0, width // 16, blk, 0)

        fire_in(0, 0)

        @pl.when(n_mine > 1)
        def _():
            fire_in(1, 1)

        def pair_body(tt, carry):
            for b in range(2):
                t = tt * 2 + b

                @pl.when(t < n_mine)
                def _(t=t, b=b):
                    pltpu.make_async_copy(tab_hbm.at[:, pl.ds(0, TRANS_W)],
                                          in_vs[b], isems[b]).wait()

                    @pl.when(t >= 2)
                    def _(b=b):
                        pltpu.make_async_copy(
                            out_vs[b],
                            out_hbm.at[pl.ds(0, TRANS_W * EMB_DIM)],
                            osems[b]).wait()

                    transpose_chunk(b, TRANS_W)
                    pltpu.async_copy(
                        out_vs[b],
                        out_hbm.at[pl.ds(c0_of(t) * EMB_DIM,
                                         TRANS_W * EMB_DIM)],
                        osems[b])

                    @pl.when(t + 2 < n_mine)
                    def _(t=t, b=b):
                        fire_in(t + 2, b)

            return carry

        lax.fori_loop(0, (n_mine + 1) // 2, pair_body, 0)
        for b in range(2):
            @pl.when(n_mine > b)
            def _(b=b):
                pltpu.make_async_copy(out_vs[b],
                                      out_hbm.at[pl.ds(0, TRANS_W * EMB_DIM)],
                                      osems[b]).wait()

        @pl.when(wid == nw - 1)
        def _():
            if rem:
                pltpu.sync_copy(tab_hbm.at[:, pl.ds(n_full * TRANS_W, rem)],
                                in_v0.at[:, pl.ds(0, rem)])
                transpose_chunk(0, rem)
                pltpu.sync_copy(
                    out_v0.at[pl.ds(0, rem * EMB_DIM)],
                    out_hbm.at[pl.ds(n_full * TRANS_W * EMB_DIM,
                                     rem * EMB_DIM)])
            pltpu.sync_copy(tail_hbm, tail_v)
            for r in range(128):
                out_v0[pl.ds(r * EMB_DIM, EMB_DIM)] = tail_v[r]
            pltpu.sync_copy(out_v0.at[pl.ds(0, 128 * EMB_DIM)],
                            out_hbm.at[pl.ds(tail_start * EMB_DIM,
                                             128 * EMB_DIM)])

    return k(tab_t, tail128)


def _sc_mean_pool(x2, table, batch):
    info = plsc.get_sparse_core_info()
    nw = info.num_cores * info.num_subcores
    rows_per_w = batch // nw
    chunks = rows_per_w // ROWS_PER_CHUNK
    mesh = plsc.VectorSubcoreMesh(core_axis_name="c", subcore_axis_name="s")

    @functools.partial(
        pl.kernel,
        out_type=jax.ShapeDtypeStruct((batch, EMB_DIM), jnp.float32),
        mesh=mesh,
        scratch_types=[
            pltpu.VMEM((IDXROWS_PER_CHUNK, IDX_MINOR), jnp.int32),
            pltpu.VMEM((IDXROWS_PER_CHUNK, IDX_MINOR), jnp.int32),
            pltpu.VMEM((IDXROWS_PER_CHUNK, IDX_MINOR), jnp.int32),
            pltpu.VMEM((IDXROWS_PER_CHUNK, IDX_MINOR, EMB_DIM), jnp.float32),
            pltpu.VMEM((IDXROWS_PER_CHUNK, IDX_MINOR, EMB_DIM), jnp.float32),
            pltpu.VMEM((IDXROWS_PER_CHUNK, IDX_MINOR, EMB_DIM), jnp.float32),
            pltpu.VMEM((512, EMB_DIM), jnp.float32),
            pltpu.SemaphoreType.DMA,
            pltpu.SemaphoreType.DMA,
            pltpu.SemaphoreType.DMA,
            pltpu.SemaphoreType.DMA,
            pltpu.SemaphoreType.DMA,
            pltpu.SemaphoreType.DMA,
        ],
        compiler_params=pltpu.CompilerParams(use_tc_tiling_on_sc=False),
    )
    def k(x_hbm, tab_hbm, out_hbm, idx_v0, idx_v1, idx_v2,
          rows_v0, rows_v1, rows_v2, means_v,
          isem0, isem1, isem2, gsem0, gsem1, gsem2):
        wid = lax.axis_index("s") * info.num_cores + lax.axis_index("c")
        idxrow_base = wid * (chunks * IDXROWS_PER_CHUNK)
        brow_base = wid * rows_per_w
        inv = jnp.float32(1.0 / HIST)
        idx_vs = (idx_v0, idx_v1, idx_v2)
        rows_vs = (rows_v0, rows_v1, rows_v2)
        isems = (isem0, isem1, isem2)
        gsems = (gsem0, gsem1, gsem2)

        def fire_idx(g, b):
            pltpu.async_copy(
                x_hbm.at[pl.ds(idxrow_base + g * IDXROWS_PER_CHUNK,
                               IDXROWS_PER_CHUNK)],
                idx_vs[b], isems[b])

        def wait_idx(b):
            pltpu.make_async_copy(
                x_hbm.at[pl.ds(0, IDXROWS_PER_CHUNK)],
                idx_vs[b], isems[b]).wait()

        def fire_gathers(b):
            for j in range(IDXROWS_PER_CHUNK):
                pltpu.async_copy(tab_hbm.at[idx_vs[b].at[j]],
                                 rows_vs[b].at[j], gsems[b])

        def drain_gathers(b):
            for j in range(IDXROWS_PER_CHUNK):
                pltpu.make_async_copy(tab_hbm.at[idx_vs[b].at[j]],
                                      rows_vs[b].at[j], gsems[b]).wait()

        def reduce_chunk(g, b):
            rows_v = rows_vs[b]

            def accum(j, k0, n8, acc):
                def body(i, c, j=j, k0=k0):
                    a0, a1, a2, a3 = c
                    kk = k0 + i * 8
                    a0 = a0 + rows_v[j, kk]
                    a1 = a1 + rows_v[j, kk + 1]
                    a2 = a2 + rows_v[j, kk + 2]
                    a3 = a3 + rows_v[j, kk + 3]
                    a0 = a0 + rows_v[j, kk + 4]
                    a1 = a1 + rows_v[j, kk + 5]
                    a2 = a2 + rows_v[j, kk + 6]
                    a3 = a3 + rows_v[j, kk + 7]
                    return a0, a1, a2, a3

                return lax.fori_loop(0, n8, body, acc)

            z = jnp.zeros((EMB_DIM,), jnp.float32)
            zz = (z, z, z, z)
            base = g * ROWS_PER_CHUNK
            for rp in range(ROWS_PER_CHUNK // 2):
                j0 = 5 * rp
                acc = accum(j0, 0, 10, zz)
                acc = accum(j0 + 1, 0, 10, acc)
                a0, a1, a2, a3 = accum(j0 + 2, 0, 5, acc)
                means_v[base + 2 * rp] = ((a0 + a1) + (a2 + a3)) * inv
                acc = accum(j0 + 2, 40, 5, zz)
                acc = accum(j0 + 3, 0, 10, acc)
                a0, a1, a2, a3 = accum(j0 + 4, 0, 10, acc)
                means_v[base + 2 * rp + 1] = ((a0 + a1) + (a2 + a3)) * inv

        fire_idx(0, 0)
        wait_idx(0)
        fire_gathers(0)
        fire_idx(1, 1)

        def tri_body(u, carry):
            for s in range(3):
                t = u * 3 + s
                b = s

                @pl.when(t < chunks)
                def _(t=t, b=b, s=s):
                    @pl.when(t + 2 < chunks)
                    def _(t=t, s=s):
                        fire_idx(t + 2, (s + 2) % 3)

                    @pl.when(t + 1 < chunks)
                    def _(t=t, s=s):
                        wait_idx((s + 1) % 3)
                        fire_gathers((s + 1) % 3)

                    drain_gathers(b)
                    reduce_chunk(t, b)

            return carry

        lax.fori_loop(0, (chunks + 2) // 3, tri_body, 0)
        pltpu.sync_copy(means_v, out_hbm.at[pl.ds(brow_base, rows_per_w)])

    return k(x2, table)


def _tc_linear(m, w_t, b):
    batch = m.shape[0]
    blk = 2048

    def body(m_ref, w_ref, b_ref, o_ref):
        o_ref[...] = jnp.dot(m_ref[...], w_ref[...],
                             preferred_element_type=jnp.float32) + b_ref[...]

    return pl.pallas_call(
        body,
        grid=(batch // blk,),
        in_specs=[
            pl.BlockSpec((blk, EMB_DIM), lambda i: (i, 0)),
            pl.BlockSpec((EMB_DIM, 2), lambda i: (0, 0)),
            pl.BlockSpec((1, 2), lambda i: (0, 0)),
        ],
        out_specs=pl.BlockSpec((blk, 2), lambda i: (i, 0)),
        out_shape=jax.ShapeDtypeStruct((batch, 2), jnp.float32),
    )(m, w_t, b.reshape(1, 2))


def kernel(x, emb_table, fc1_w, fc1_b):
    batch, hist = x.shape
    x2 = x.reshape(batch * hist // IDX_MINOR, IDX_MINOR).astype(jnp.int32)
    tail128 = lax.slice(emb_table, (emb_table.shape[0] - 128, 0),
                        emb_table.shape)
    tab_flat = _sc_detile_table(emb_table.T, tail128)
    tab2 = tab_flat.reshape(emb_table.shape)
    means = _sc_mean_pool(x2, tab2, batch)
    return _tc_linear(means, fc1_w.T, fc1_b)


# speedup vs baseline: 1.0004x; 1.0004x over previous
import functools

import jax
import jax.numpy as jnp
from jax import lax
from jax.experimental import pallas as pl
from jax.experimental.pallas import tpu as pltpu
from jax.experimental.pallas import tpu_sc as plsc

EMB_DIM = 16
HIST = 200
IDX_MINOR = 80
ROWS_PER_CHUNK = 8
IDXROWS_PER_CHUNK = ROWS_PER_CHUNK * HIST // IDX_MINOR
TRANS_W = 1024


def _sc_detile_table(tab_t, tail128):
    vocab = tab_t.shape[1]
    aligned_v = vocab // 128 * 128
    n_full = aligned_v // TRANS_W
    rem = aligned_v - n_full * TRANS_W
    tail_start = vocab - 128
    info = plsc.get_sparse_core_info()
    nw = info.num_cores * info.num_subcores
    mesh = plsc.VectorSubcoreMesh(core_axis_name="c", subcore_axis_name="s")

    @functools.partial(
        pl.kernel,
        out_type=jax.ShapeDtypeStruct((vocab * EMB_DIM,), jnp.float32),
        mesh=mesh,
        scratch_types=[
            pltpu.VMEM((EMB_DIM, TRANS_W), jnp.float32),
            pltpu.VMEM((EMB_DIM, TRANS_W), jnp.float32),
            pltpu.VMEM((TRANS_W * EMB_DIM,), jnp.float32),
            pltpu.VMEM((TRANS_W * EMB_DIM,), jnp.float32),
            pltpu.VMEM((128, EMB_DIM), jnp.float32),
            pltpu.SemaphoreType.DMA,
            pltpu.SemaphoreType.DMA,
            pltpu.SemaphoreType.DMA,
            pltpu.SemaphoreType.DMA,
        ],
        compiler_params=pltpu.CompilerParams(use_tc_tiling_on_sc=True,
                                             needs_layout_passes=False),
    )
    def k(tab_hbm, tail_hbm, out_hbm, in_v0, in_v1, out_v0, out_v1, tail_v,
          isem0, isem1, osem0, osem1):
        wid = lax.axis_index("s") * info.num_cores + lax.axis_index("c")
        rows_i = lax.iota(jnp.int32, 16)
        in_vs = (in_v0, in_v1)
        out_vs = (out_v0, out_v1)
        isems = (isem0, isem1)
        osems = (osem0, osem1)
        n_mine = (n_full - 1 - wid) // nw + 1

        def c0_of(t):
            return (wid + t * nw) * TRANS_W

        def fire_in(t, b):
            pltpu.async_copy(tab_hbm.at[:, pl.ds(c0_of(t), TRANS_W)],
                             in_vs[b], isems[b])

        def transpose_chunk(b, width):
            def blk(i, carry, b=b):
                c0 = i * 16
                rot = rows_i
                for _ in range(16):
                    colv = rot + c0
                    gvals = plsc.load_gather(in_vs[b], [rows_i, colv])
                    sidx = jnp.left_shift(colv, 4) + rows_i
                    plsc.store_scatter(out_vs[b], [sidx], gvals)
                    rot = jnp.bitwise_and(rot + 1, 15)
                return carry

            lax.fori_loop(0, width // 16, blk, 0)

        fire_in(0, 0)

        @pl.when(n_mine > 1)
        def _():
            fire_in(1, 1)

        def pair_body(tt, carry):
            for b in range(2):
                t = tt * 2 + b

                @pl.when(t < n_mine)
                def _(t=t, b=b):
                    pltpu.make_async_copy(tab_hbm.at[:, pl.ds(0, TRANS_W)],
                                          in_vs[b], isems[b]).wait()

                    @pl.when(t >= 2)
                    def _(b=b):
                        pltpu.make_async_copy(
                            out_vs[b],
                            out_hbm.at[pl.ds(0, TRANS_W * EMB_DIM)],
                            osems[b]).wait()

                    transpose_chunk(b, TRANS_W)
                    pltpu.async_copy(
                        out_vs[b],
                        out_hbm.at[pl.ds(c0_of(t) * EMB_DIM,
                                         TRANS_W * EMB_DIM)],
                        osems[b])

                    @pl.when(t + 2 < n_mine)
                    def _(t=t, b=b):
                        fire_in(t + 2, b)

            return carry

        lax.fori_loop(0, (n_mine + 1) // 2, pair_body, 0)
        for b in range(2):
            @pl.when(n_mine > b)
            def _(b=b):
                pltpu.make_async_copy(out_vs[b],
                                      out_hbm.at[pl.ds(0, TRANS_W * EMB_DIM)],
                                      osems[b]).wait()

        @pl.when(wid == nw - 1)
        def _():
            if rem:
                pltpu.sync_copy(tab_hbm.at[:, pl.ds(n_full * TRANS_W, rem)],
                                in_v0.at[:, pl.ds(0, rem)])
                transpose_chunk(0, rem)
                pltpu.sync_copy(
                    out_v0.at[pl.ds(0, rem * EMB_DIM)],
                    out_hbm.at[pl.ds(n_full * TRANS_W * EMB_DIM,
                                     rem * EMB_DIM)])
            pltpu.sync_copy(tail_hbm, tail_v)
            for r in range(128):
                out_v0[pl.ds(r * EMB_DIM, EMB_DIM)] = tail_v[r]
            pltpu.sync_copy(out_v0.at[pl.ds(0, 128 * EMB_DIM)],
                            out_hbm.at[pl.ds(tail_start * EMB_DIM,
                                             128 * EMB_DIM)])

    return k(tab_t, tail128)


def _sc_mean_pool(x2, table, batch):
    info = plsc.get_sparse_core_info()
    nw = info.num_cores * info.num_subcores
    rows_per_w = batch // nw
    chunks = rows_per_w // ROWS_PER_CHUNK
    mesh = plsc.VectorSubcoreMesh(core_axis_name="c", subcore_axis_name="s")

    @functools.partial(
        pl.kernel,
        out_type=jax.ShapeDtypeStruct((batch, EMB_DIM), jnp.float32),
        mesh=mesh,
        scratch_types=[
            pltpu.VMEM((IDXROWS_PER_CHUNK, IDX_MINOR), jnp.int32),
            pltpu.VMEM((IDXROWS_PER_CHUNK, IDX_MINOR), jnp.int32),
            pltpu.VMEM((IDXROWS_PER_CHUNK, IDX_MINOR), jnp.int32),
            pltpu.VMEM((IDXROWS_PER_CHUNK, IDX_MINOR, EMB_DIM), jnp.float32),
            pltpu.VMEM((IDXROWS_PER_CHUNK, IDX_MINOR, EMB_DIM), jnp.float32),
            pltpu.VMEM((IDXROWS_PER_CHUNK, IDX_MINOR, EMB_DIM), jnp.float32),
            pltpu.VMEM((512, EMB_DIM), jnp.float32),
            pltpu.SemaphoreType.DMA,
            pltpu.SemaphoreType.DMA,
            pltpu.SemaphoreType.DMA,
            pltpu.SemaphoreType.DMA,
            pltpu.SemaphoreType.DMA,
            pltpu.SemaphoreType.DMA,
        ],
        compiler_params=pltpu.CompilerParams(use_tc_tiling_on_sc=False),
    )
    def k(x_hbm, tab_hbm, out_hbm, idx_v0, idx_v1, idx_v2,
          rows_v0, rows_v1, rows_v2, means_v,
          isem0, isem1, isem2, gsem0, gsem1, gsem2):
        wid = lax.axis_index("s") * info.num_cores + lax.axis_index("c")
        idxrow_base = wid * (chunks * IDXROWS_PER_CHUNK)
        brow_base = wid * rows_per_w
        inv = jnp.float32(1.0 / HIST)
        idx_vs = (idx_v0, idx_v1, idx_v2)
        rows_vs = (rows_v0, rows_v1, rows_v2)
        isems = (isem0, isem1, isem2)
        gsems = (gsem0, gsem1, gsem2)

        def fire_idx(g, b):
            pltpu.async_copy(
                x_hbm.at[pl.ds(idxrow_base + g * IDXROWS_PER_CHUNK,
                               IDXROWS_PER_CHUNK)],
                idx_vs[b], isems[b])

        def wait_idx(b):
            pltpu.make_async_copy(
                x_hbm.at[pl.ds(0, IDXROWS_PER_CHUNK)],
                idx_vs[b], isems[b]).wait()

        def fire_gathers(b):
            for j in range(IDXROWS_PER_CHUNK):
                pltpu.async_copy(tab_hbm.at[idx_vs[b].at[j]],
                                 rows_vs[b].at[j], gsems[b])

        def drain_gathers(b):
            for j in range(IDXROWS_PER_CHUNK):
                pltpu.make_async_copy(tab_hbm.at[idx_vs[b].at[j]],
                                      rows_vs[b].at[j], gsems[b]).wait()

        def reduce_chunk(g, b):
            rows_v = rows_vs[b]

            def accum(j, k0, n8, acc):
                def body(i, c, j=j, k0=k0):
                    a0, a1, a2, a3 = c
                    kk = k0 + i * 8
                    a0 = a0 + rows_v[j, kk]
                    a1 = a1 + rows_v[j, kk + 1]
                    a2 = a2 + rows_v[j, kk + 2]
                    a3 = a3 + rows_v[j, kk + 3]
                    a0 = a0 + rows_v[j, kk + 4]
                    a1 = a1 + rows_v[j, kk + 5]
                    a2 = a2 + rows_v[j, kk + 6]
                    a3 = a3 + rows_v[j, kk + 7]
                    return a0, a1, a2, a3

                return lax.fori_loop(0, n8, body, acc)

            z = jnp.zeros((EMB_DIM,), jnp.float32)
            zz = (z, z, z, z)
            base = g * ROWS_PER_CHUNK
            for rp in range(ROWS_PER_CHUNK // 2):
                j0 = 5 * rp
                acc = accum(j0, 0, 10, zz)
                acc = accum(j0 + 1, 0, 10, acc)
                a0, a1, a2, a3 = accum(j0 + 2, 0, 5, acc)
                means_v[base + 2 * rp] = ((a0 + a1) + (a2 + a3)) * inv
                acc = accum(j0 + 2, 40, 5, zz)
                acc = accum(j0 + 3, 0, 10, acc)
                a0, a1, a2, a3 = accum(j0 + 4, 0, 10, acc)
                means_v[base + 2 * rp + 1] = ((a0 + a1) + (a2 + a3)) * inv

        fire_idx(0, 0)
        wait_idx(0)
        fire_gathers(0)
        fire_idx(1, 1)

        def tri_body(u, carry):
            for s in range(3):
                t = u * 3 + s
                b = s

                @pl.when(t < chunks)
                def _(t=t, b=b, s=s):
                    @pl.when(t + 2 < chunks)
                    def _(t=t, s=s):
                        fire_idx(t + 2, (s + 2) % 3)

                    @pl.when(t + 1 < chunks)
                    def _(t=t, s=s):
                        wait_idx((s + 1) % 3)
                        fire_gathers((s + 1) % 3)

                    drain_gathers(b)
                    reduce_chunk(t, b)

            return carry

        lax.fori_loop(0, (chunks + 2) // 3, tri_body, 0)
        pltpu.sync_copy(means_v, out_hbm.at[pl.ds(brow_base, rows_per_w)])

    return k(x2, table)


def _tc_linear(m, w_t, b):
    batch = m.shape[0]
    blk = 2048

    def body(m_ref, w_ref, b_ref, o_ref):
        o_ref[...] = jnp.dot(m_ref[...], w_ref[...],
                             preferred_element_type=jnp.float32) + b_ref[...]

    return pl.pallas_call(
        body,
        grid=(batch // blk,),
        in_specs=[
            pl.BlockSpec((blk, EMB_DIM), lambda i: (i, 0)),
            pl.BlockSpec((EMB_DIM, 2), lambda i: (0, 0)),
            pl.BlockSpec((1, 2), lambda i: (0, 0)),
        ],
        out_specs=pl.BlockSpec((blk, 2), lambda i: (i, 0)),
        out_shape=jax.ShapeDtypeStruct((batch, 2), jnp.float32),
    )(m, w_t, b.reshape(1, 2))


def kernel(x, emb_table, fc1_w, fc1_b):
    batch, hist = x.shape
    x2 = x.reshape(batch * hist // IDX_MINOR, IDX_MINOR).astype(jnp.int32)
    tail128 = lax.slice(emb_table, (emb_table.shape[0] - 128, 0),
                        emb_table.shape)
    tab_flat = _sc_detile_table(emb_table.T, tail128)
    tab2 = tab_flat.reshape(emb_table.shape)
    means = _sc_mean_pool(x2, tab2, batch)
    return _tc_linear(means, fc1_w.T, fc1_b)


# speedup vs baseline: 1.2027x; 1.2022x over previous
import functools

import jax
import jax.numpy as jnp
from jax import lax
from jax.experimental import pallas as pl
from jax.experimental.pallas import tpu as pltpu
from jax.experimental.pallas import tpu_sc as plsc

EMB_DIM = 16
HIST = 200
IDX_MINOR = 80
ROWS_PER_CHUNK = 8
IDXROWS_PER_CHUNK = ROWS_PER_CHUNK * HIST // IDX_MINOR
TRANS_W = 1024


def _sc_detile_table(tab_t, tail128):
    vocab = tab_t.shape[1]
    aligned_v = vocab // 128 * 128
    n_full = aligned_v // TRANS_W
    rem = aligned_v - n_full * TRANS_W
    tail_start = vocab - 128
    info = plsc.get_sparse_core_info()
    nw = info.num_cores * info.num_subcores
    mesh = plsc.VectorSubcoreMesh(core_axis_name="c", subcore_axis_name="s")

    @functools.partial(
        pl.kernel,
        out_type=jax.ShapeDtypeStruct((vocab * EMB_DIM,), jnp.float32),
        mesh=mesh,
        scratch_types=[
            pltpu.VMEM((EMB_DIM, TRANS_W), jnp.float32),
            pltpu.VMEM((EMB_DIM, TRANS_W), jnp.float32),
            pltpu.VMEM((TRANS_W * EMB_DIM,), jnp.float32),
            pltpu.VMEM((TRANS_W * EMB_DIM,), jnp.float32),
            pltpu.VMEM((128, EMB_DIM), jnp.float32),
            pltpu.SemaphoreType.DMA,
            pltpu.SemaphoreType.DMA,
            pltpu.SemaphoreType.DMA,
            pltpu.SemaphoreType.DMA,
        ],
        compiler_params=pltpu.CompilerParams(use_tc_tiling_on_sc=True,
                                             needs_layout_passes=False),
    )
    def k(tab_hbm, tail_hbm, out_hbm, in_v0, in_v1, out_v0, out_v1, tail_v,
          isem0, isem1, osem0, osem1):
        wid = lax.axis_index("s") * info.num_cores + lax.axis_index("c")
        rows_i = lax.iota(jnp.int32, 16)
        in_vs = (in_v0, in_v1)
        out_vs = (out_v0, out_v1)
        isems = (isem0, isem1)
        osems = (osem0, osem1)
        n_mine = (n_full - 1 - wid) // nw + 1

        def c0_of(t):
            return (wid + t * nw) * TRANS_W

        def fire_in(t, b):
            pltpu.async_copy(tab_hbm.at[:, pl.ds(c0_of(t), TRANS_W)],
                             in_vs[b], isems[b])

        def transpose_chunk(b, width):
            def blk(i, carry, b=b):
                c0 = i * 16
                rot = rows_i
                for _ in range(16):
                    colv = rot + c0
                    gvals = plsc.load_gather(in_vs[b], [rows_i, colv])
                    sidx = jnp.left_shift(colv, 4) + rows_i
                    plsc.store_scatter(out_vs[b], [sidx], gvals)
                    rot = jnp.bitwise_and(rot + 1, 15)
                return carry

            lax.fori_loop(0, width // 16, blk, 0)

        fire_in(0, 0)

        @pl.when(n_mine > 1)
        def _():
            fire_in(1, 1)

        def pair_body(tt, carry):
            for b in range(2):
                t = tt * 2 + b

                @pl.when(t < n_mine)
                def _(t=t, b=b):
                    pltpu.make_async_copy(tab_hbm.at[:, pl.ds(0, TRANS_W)],
                                          in_vs[b], isems[b]).wait()

                    @pl.when(t >= 2)
                    def _(b=b):
                        pltpu.make_async_copy(
                            out_vs[b],
                            out_hbm.at[pl.ds(0, TRANS_W * EMB_DIM)],
                            osems[b]).wait()

                    transpose_chunk(b, TRANS_W)
                    pltpu.async_copy(
                        out_vs[b],
                        out_hbm.at[pl.ds(c0_of(t) * EMB_DIM,
                                         TRANS_W * EMB_DIM)],
                        osems[b])

                    @pl.when(t + 2 < n_mine)
                    def _(t=t, b=b):
                        fire_in(t + 2, b)

            return carry

        lax.fori_loop(0, (n_mine + 1) // 2, pair_body, 0)
        for b in range(2):
            @pl.when(n_mine > b)
            def _(b=b):
                pltpu.make_async_copy(out_vs[b],
                                      out_hbm.at[pl.ds(0, TRANS_W * EMB_DIM)],
                                      osems[b]).wait()

        @pl.when(wid == nw - 1)
        def _():
            if rem:
                pltpu.sync_copy(tab_hbm.at[:, pl.ds(n_full * TRANS_W, rem)],
                                in_v0.at[:, pl.ds(0, rem)])
                transpose_chunk(0, rem)
                pltpu.sync_copy(
                    out_v0.at[pl.ds(0, rem * EMB_DIM)],
                    out_hbm.at[pl.ds(n_full * TRANS_W * EMB_DIM,
                                     rem * EMB_DIM)])
            pltpu.sync_copy(tail_hbm, tail_v)
            for r in range(128):
                out_v0[pl.ds(r * EMB_DIM, EMB_DIM)] = tail_v[r]
            pltpu.sync_copy(out_v0.at[pl.ds(0, 128 * EMB_DIM)],
                            out_hbm.at[pl.ds(tail_start * EMB_DIM,
                                             128 * EMB_DIM)])

    return k(tab_t, tail128)


def _sc_mean_pool(x2, table, batch):
    info = plsc.get_sparse_core_info()
    nw = info.num_cores * info.num_subcores
    rows_per_w = batch // nw
    chunks = rows_per_w // ROWS_PER_CHUNK
    mesh = plsc.VectorSubcoreMesh(core_axis_name="c", subcore_axis_name="s")

    @functools.partial(
        pl.kernel,
        out_type=jax.ShapeDtypeStruct((batch, EMB_DIM), jnp.float32),
        mesh=mesh,
        scratch_types=[
            pltpu.VMEM((IDXROWS_PER_CHUNK, IDX_MINOR), jnp.int32),
            pltpu.VMEM((IDXROWS_PER_CHUNK, IDX_MINOR), jnp.int32),
            pltpu.VMEM((IDXROWS_PER_CHUNK, IDX_MINOR), jnp.int32),
            pltpu.VMEM((IDXROWS_PER_CHUNK, IDX_MINOR, EMB_DIM), jnp.float32),
            pltpu.VMEM((IDXROWS_PER_CHUNK, IDX_MINOR, EMB_DIM), jnp.float32),
            pltpu.VMEM((IDXROWS_PER_CHUNK, IDX_MINOR, EMB_DIM), jnp.float32),
            pltpu.VMEM((512, EMB_DIM), jnp.float32),
            pltpu.SemaphoreType.DMA,
            pltpu.SemaphoreType.DMA,
            pltpu.SemaphoreType.DMA,
            pltpu.SemaphoreType.DMA,
            pltpu.SemaphoreType.DMA,
            pltpu.SemaphoreType.DMA,
        ],
        compiler_params=pltpu.CompilerParams(use_tc_tiling_on_sc=False),
    )
    def k(x_hbm, tab_hbm, out_hbm, idx_v0, idx_v1, idx_v2,
          rows_v0, rows_v1, rows_v2, means_v,
          isem0, isem1, isem2, gsem0, gsem1, gsem2):
        wid = lax.axis_index("s") * info.num_cores + lax.axis_index("c")
        idxrow_base = wid * (chunks * IDXROWS_PER_CHUNK)
        brow_base = wid * rows_per_w
        inv = jnp.float32(1.0 / HIST)
        idx_vs = (idx_v0, idx_v1, idx_v2)
        rows_vs = (rows_v0, rows_v1, rows_v2)
        isems = (isem0, isem1, isem2)
        gsems = (gsem0, gsem1, gsem2)

        def fire_idx(g, b):
            pltpu.async_copy(
                x_hbm.at[pl.ds(idxrow_base + g * IDXROWS_PER_CHUNK,
                               IDXROWS_PER_CHUNK)],
                idx_vs[b], isems[b])

        def wait_idx(b):
            pltpu.make_async_copy(
                x_hbm.at[pl.ds(0, IDXROWS_PER_CHUNK)],
                idx_vs[b], isems[b]).wait()

        def fire_gathers(b):
            for j in range(IDXROWS_PER_CHUNK):
                pltpu.async_copy(tab_hbm.at[idx_vs[b].at[j]],
                                 rows_vs[b].at[j], gsems[b])

        def drain_gathers(b):
            for j in range(IDXROWS_PER_CHUNK):
                pltpu.make_async_copy(tab_hbm.at[idx_vs[b].at[j]],
                                      rows_vs[b].at[j], gsems[b]).wait()

        def reduce_chunk(g, b):
            rows_v = rows_vs[b]

            def accum(j, k0, n4, acc):
                def body(i, c, j=j, k0=k0):
                    a0, a1 = c
                    kk = k0 + i * 4
                    a0 = a0 + rows_v[j, kk]
                    a1 = a1 + rows_v[j, kk + 1]
                    a0 = a0 + rows_v[j, kk + 2]
                    a1 = a1 + rows_v[j, kk + 3]
                    return a0, a1

                return lax.fori_loop(0, n4, body, acc)

            z = jnp.zeros((EMB_DIM,), jnp.float32)
            base = g * ROWS_PER_CHUNK
            for rp in range(ROWS_PER_CHUNK // 2):
                j0 = 5 * rp
                acc = accum(j0, 0, 20, (z, z))
                acc = accum(j0 + 1, 0, 20, acc)
                a0, a1 = accum(j0 + 2, 0, 10, acc)
                means_v[base + 2 * rp] = (a0 + a1) * inv
                acc = accum(j0 + 2, 40, 10, (z, z))
                acc = accum(j0 + 3, 0, 20, acc)
                a0, a1 = accum(j0 + 4, 0, 20, acc)
                means_v[base + 2 * rp + 1] = (a0 + a1) * inv

        fire_idx(0, 0)
        wait_idx(0)
        fire_gathers(0)
        fire_idx(1, 1)

        def tri_body(u, carry):
            for s in range(3):
                t = u * 3 + s
                b = s

                @pl.when(t < chunks)
                def _(t=t, b=b, s=s):
                    @pl.when(t + 2 < chunks)
                    def _(t=t, s=s):
                        fire_idx(t + 2, (s + 2) % 3)

                    @pl.when(t + 1 < chunks)
                    def _(t=t, s=s):
                        wait_idx((s + 1) % 3)
                        fire_gathers((s + 1) % 3)

                    drain_gathers(b)
                    reduce_chunk(t, b)

            return carry

        lax.fori_loop(0, (chunks + 2) // 3, tri_body, 0)
        pltpu.sync_copy(means_v, out_hbm.at[pl.ds(brow_base, rows_per_w)])

    return k(x2, table)


def _tc_linear(m, w_t, b):
    batch = m.shape[0]
    blk = 2048

    def body(m_ref, w_ref, b_ref, o_ref):
        o_ref[...] = jnp.dot(m_ref[...], w_ref[...],
                             preferred_element_type=jnp.float32) + b_ref[...]

    return pl.pallas_call(
        body,
        grid=(batch // blk,),
        in_specs=[
            pl.BlockSpec((blk, EMB_DIM), lambda i: (i, 0)),
            pl.BlockSpec((EMB_DIM, 2), lambda i: (0, 0)),
            pl.BlockSpec((1, 2), lambda i: (0, 0)),
        ],
        out_specs=pl.BlockSpec((blk, 2), lambda i: (i, 0)),
        out_shape=jax.ShapeDtypeStruct((batch, 2), jnp.float32),
    )(m, w_t, b.reshape(1, 2))


def kernel(x, emb_table, fc1_w, fc1_b):
    batch, hist = x.shape
    x2 = x.reshape(batch * hist // IDX_MINOR, IDX_MINOR).astype(jnp.int32)
    tail128 = lax.slice(emb_table, (emb_table.shape[0] - 128, 0),
                        emb_table.shape)
    tab_flat = _sc_detile_table(emb_table.T, tail128)
    tab2 = tab_flat.reshape(emb_table.shape)
    means = _sc_mean_pool(x2, tab2, batch)
    return _tc_linear(means, fc1_w.T, fc1_b)


# speedup vs baseline: 1.2074x; 1.0039x over previous
import functools

import jax
import jax.numpy as jnp
from jax import lax
from jax.experimental import pallas as pl
from jax.experimental.pallas import tpu as pltpu
from jax.experimental.pallas import tpu_sc as plsc

EMB_DIM = 16
HIST = 200
IDX_MINOR = 80
ROWS_PER_CHUNK = 8
IDXROWS_PER_CHUNK = ROWS_PER_CHUNK * HIST // IDX_MINOR
TRANS_W = 1024


def _sc_detile_table(tab_t, tail128):
    vocab = tab_t.shape[1]
    aligned_v = vocab // 128 * 128
    n_full = aligned_v // TRANS_W
    rem = aligned_v - n_full * TRANS_W
    tail_start = vocab - 128
    info = plsc.get_sparse_core_info()
    nw = info.num_cores * info.num_subcores
    mesh = plsc.VectorSubcoreMesh(core_axis_name="c", subcore_axis_name="s")

    @functools.partial(
        pl.kernel,
        out_type=jax.ShapeDtypeStruct((vocab * EMB_DIM,), jnp.float32),
        mesh=mesh,
        scratch_types=[
            pltpu.VMEM((EMB_DIM, TRANS_W), jnp.float32),
            pltpu.VMEM((EMB_DIM, TRANS_W), jnp.float32),
            pltpu.VMEM((TRANS_W * EMB_DIM,), jnp.float32),
            pltpu.VMEM((TRANS_W * EMB_DIM,), jnp.float32),
            pltpu.VMEM((128, EMB_DIM), jnp.float32),
            pltpu.SemaphoreType.DMA,
            pltpu.SemaphoreType.DMA,
            pltpu.SemaphoreType.DMA,
            pltpu.SemaphoreType.DMA,
        ],
        compiler_params=pltpu.CompilerParams(use_tc_tiling_on_sc=True,
                                             needs_layout_passes=False),
    )
    def k(tab_hbm, tail_hbm, out_hbm, in_v0, in_v1, out_v0, out_v1, tail_v,
          isem0, isem1, osem0, osem1):
        wid = lax.axis_index("s") * info.num_cores + lax.axis_index("c")
        rows_i = lax.iota(jnp.int32, 16)
        in_vs = (in_v0, in_v1)
        out_vs = (out_v0, out_v1)
        isems = (isem0, isem1)
        osems = (osem0, osem1)
        n_mine = (n_full - 1 - wid) // nw + 1

        def c0_of(t):
            return (wid + t * nw) * TRANS_W

        def fire_in(t, b):
            pltpu.async_copy(tab_hbm.at[:, pl.ds(c0_of(t), TRANS_W)],
                             in_vs[b], isems[b])

        def transpose_chunk(b, width):
            def blk(i, carry, b=b):
                c0 = i * 16
                rot = rows_i
                for _ in range(16):
                    colv = rot + c0
                    gvals = plsc.load_gather(in_vs[b], [rows_i, colv])
                    sidx = jnp.left_shift(colv, 4) + rows_i
                    plsc.store_scatter(out_vs[b], [sidx], gvals)
                    rot = jnp.bitwise_and(rot + 1, 15)
                return carry

            lax.fori_loop(0, width // 16, blk, 0)

        fire_in(0, 0)

        @pl.when(n_mine > 1)
        def _():
            fire_in(1, 1)

        def pair_body(tt, carry):
            for b in range(2):
                t = tt * 2 + b

                @pl.when(t < n_mine)
                def _(t=t, b=b):
                    pltpu.make_async_copy(tab_hbm.at[:, pl.ds(0, TRANS_W)],
                                          in_vs[b], isems[b]).wait()

                    @pl.when(t >= 2)
                    def _(b=b):
                        pltpu.make_async_copy(
                            out_vs[b],
                            out_hbm.at[pl.ds(0, TRANS_W * EMB_DIM)],
                            osems[b]).wait()

                    transpose_chunk(b, TRANS_W)
                    pltpu.async_copy(
                        out_vs[b],
                        out_hbm.at[pl.ds(c0_of(t) * EMB_DIM,
                                         TRANS_W * EMB_DIM)],
                        osems[b])

                    @pl.when(t + 2 < n_mine)
                    def _(t=t, b=b):
                        fire_in(t + 2, b)

            return carry

        lax.fori_loop(0, (n_mine + 1) // 2, pair_body, 0)
        for b in range(2):
            @pl.when(n_mine > b)
            def _(b=b):
                pltpu.make_async_copy(out_vs[b],
                                      out_hbm.at[pl.ds(0, TRANS_W * EMB_DIM)],
                                      osems[b]).wait()

        @pl.when(wid == nw - 1)
        def _():
            if rem:
                pltpu.sync_copy(tab_hbm.at[:, pl.ds(n_full * TRANS_W, rem)],
                                in_v0.at[:, pl.ds(0, rem)])
                transpose_chunk(0, rem)
                pltpu.sync_copy(
                    out_v0.at[pl.ds(0, rem * EMB_DIM)],
                    out_hbm.at[pl.ds(n_full * TRANS_W * EMB_DIM,
                                     rem * EMB_DIM)])
            pltpu.sync_copy(tail_hbm, tail_v)
            for r in range(128):
                out_v0[pl.ds(r * EMB_DIM, EMB_DIM)] = tail_v[r]
            pltpu.sync_copy(out_v0.at[pl.ds(0, 128 * EMB_DIM)],
                            out_hbm.at[pl.ds(tail_start * EMB_DIM,
                                             128 * EMB_DIM)])

    return k(tab_t, tail128)


def _sc_mean_pool(x2, table, batch):
    info = plsc.get_sparse_core_info()
    nw = info.num_cores * info.num_subcores
    rows_per_w = batch // nw
    chunks = rows_per_w // ROWS_PER_CHUNK
    mesh = plsc.VectorSubcoreMesh(core_axis_name="c", subcore_axis_name="s")

    @functools.partial(
        pl.kernel,
        out_type=jax.ShapeDtypeStruct((batch, EMB_DIM), jnp.float32),
        mesh=mesh,
        scratch_types=[
            pltpu.VMEM((ROWS_PER_CHUNK * HIST,), jnp.int32),
            pltpu.VMEM((ROWS_PER_CHUNK * HIST,), jnp.int32),
            pltpu.VMEM((ROWS_PER_CHUNK * HIST,), jnp.int32),
            pltpu.VMEM((ROWS_PER_CHUNK * HIST, EMB_DIM), jnp.float32),
            pltpu.VMEM((ROWS_PER_CHUNK * HIST, EMB_DIM), jnp.float32),
            pltpu.VMEM((ROWS_PER_CHUNK * HIST, EMB_DIM), jnp.float32),
            pltpu.VMEM((512, EMB_DIM), jnp.float32),
            pltpu.SemaphoreType.DMA,
            pltpu.SemaphoreType.DMA,
            pltpu.SemaphoreType.DMA,
            pltpu.SemaphoreType.DMA,
            pltpu.SemaphoreType.DMA,
            pltpu.SemaphoreType.DMA,
        ],
        compiler_params=pltpu.CompilerParams(use_tc_tiling_on_sc=False),
    )
    def k(x_hbm, tab_hbm, out_hbm, idx_v0, idx_v1, idx_v2,
          rows_v0, rows_v1, rows_v2, means_v,
          isem0, isem1, isem2, gsem0, gsem1, gsem2):
        wid = lax.axis_index("s") * info.num_cores + lax.axis_index("c")
        el_base = wid * (chunks * ROWS_PER_CHUNK * HIST)
        brow_base = wid * rows_per_w
        inv = jnp.float32(1.0 / HIST)
        idx_vs = (idx_v0, idx_v1, idx_v2)
        rows_vs = (rows_v0, rows_v1, rows_v2)
        isems = (isem0, isem1, isem2)
        gsems = (gsem0, gsem1, gsem2)

        nel = ROWS_PER_CHUNK * HIST

        def fire_idx(g, b):
            pltpu.async_copy(
                x_hbm.at[pl.ds(el_base + g * nel, nel)],
                idx_vs[b], isems[b])

        def wait_idx(b):
            pltpu.make_async_copy(
                x_hbm.at[pl.ds(0, nel)],
                idx_vs[b], isems[b]).wait()

        def fire_gathers(b):
            pltpu.async_copy(tab_hbm.at[idx_vs[b]], rows_vs[b], gsems[b])

        def drain_gathers(b):
            pltpu.make_async_copy(tab_hbm.at[idx_vs[b]],
                                  rows_vs[b], gsems[b]).wait()

        def reduce_chunk(g, b):
            rows_v = rows_vs[b]

            z = jnp.zeros((EMB_DIM,), jnp.float32)
            base = g * ROWS_PER_CHUNK
            for r in range(ROWS_PER_CHUNK):
                def body(i, c, r=r):
                    a0, a1 = c
                    kk = r * HIST + i * 4
                    a0 = a0 + rows_v[kk]
                    a1 = a1 + rows_v[kk + 1]
                    a0 = a0 + rows_v[kk + 2]
                    a1 = a1 + rows_v[kk + 3]
                    return a0, a1

                a0, a1 = lax.fori_loop(0, HIST // 4, body, (z, z))
                means_v[base + r] = (a0 + a1) * inv

        fire_idx(0, 0)
        wait_idx(0)
        fire_gathers(0)
        fire_idx(1, 1)

        def tri_body(u, carry):
            for s in range(3):
                t = u * 3 + s
                b = s

                @pl.when(t < chunks)
                def _(t=t, b=b, s=s):
                    @pl.when(t + 2 < chunks)
                    def _(t=t, s=s):
                        fire_idx(t + 2, (s + 2) % 3)

                    @pl.when(t + 1 < chunks)
                    def _(t=t, s=s):
                        wait_idx((s + 1) % 3)
                        fire_gathers((s + 1) % 3)

                    drain_gathers(b)
                    reduce_chunk(t, b)

            return carry

        lax.fori_loop(0, (chunks + 2) // 3, tri_body, 0)
        pltpu.sync_copy(means_v, out_hbm.at[pl.ds(brow_base, rows_per_w)])

    return k(x2, table)


def _tc_linear(m, w_t, b):
    batch = m.shape[0]
    blk = 2048

    def body(m_ref, w_ref, b_ref, o_ref):
        o_ref[...] = jnp.dot(m_ref[...], w_ref[...],
                             preferred_element_type=jnp.float32) + b_ref[...]

    return pl.pallas_call(
        body,
        grid=(batch // blk,),
        in_specs=[
            pl.BlockSpec((blk, EMB_DIM), lambda i: (i, 0)),
            pl.BlockSpec((EMB_DIM, 2), lambda i: (0, 0)),
            pl.BlockSpec((1, 2), lambda i: (0, 0)),
        ],
        out_specs=pl.BlockSpec((blk, 2), lambda i: (i, 0)),
        out_shape=jax.ShapeDtypeStruct((batch, 2), jnp.float32),
    )(m, w_t, b.reshape(1, 2))


def kernel(x, emb_table, fc1_w, fc1_b):
    batch, hist = x.shape
    x2 = x.reshape(batch * hist).astype(jnp.int32)
    tail128 = lax.slice(emb_table, (emb_table.shape[0] - 128, 0),
                        emb_table.shape)
    tab_flat = _sc_detile_table(emb_table.T, tail128)
    tab2 = tab_flat.reshape(emb_table.shape)
    means = _sc_mean_pool(x2, tab2, batch)
    return _tc_linear(means, fc1_w.T, fc1_b)
